# Initial kernel scaffold; baseline (speedup 1.0000x reference)
import jax
import jax.numpy as jnp
from jax.experimental import pallas as pl


def kernel(node_features, edge_features, senders, receivers, eW1, eb1, eW2, eb2, eW3, eb3, eg, ebt, nW1, nb1, nW2, nb2, nW3, nb3, ng, nbt):
    raise NotImplementedError("write your pallas kernel here")


# speedup vs baseline: 3.5325x; 3.5325x over previous
import functools

import jax
import jax.numpy as jnp
from jax import lax
from jax.experimental import pallas as pl
from jax.experimental.pallas import tpu as pltpu
from jax.experimental.pallas import tpu_sc as plsc

N = 10000
E = 320000
D = 128

NC = 2
NS = 16
NW = NC * NS
LANES = 16

GW = 80


def _mesh():
    return plsc.VectorSubcoreMesh(core_axis_name="c", subcore_axis_name="s")


def _pq_body(nf, w1a, w1b, b1, p_out, q_out):
    x = nf[...]
    p_out[...] = jnp.dot(x, w1a[...], preferred_element_type=jnp.float32) + b1[...]
    q_out[...] = jnp.dot(x, w1b[...], preferred_element_type=jnp.float32)


def _compute_pq(nf, w1a, w1b, b1):
    blk = 2000
    return pl.pallas_call(
        _pq_body,
        grid=(N // blk,),
        in_specs=[
            pl.BlockSpec((blk, D), lambda i: (i, 0)),
            pl.BlockSpec((D, D), lambda i: (0, 0)),
            pl.BlockSpec((D, D), lambda i: (0, 0)),
            pl.BlockSpec((1, D), lambda i: (0, 0)),
        ],
        out_specs=[pl.BlockSpec((blk, D), lambda i: (i, 0)),
                   pl.BlockSpec((blk, D), lambda i: (i, 0))],
        out_shape=[jax.ShapeDtypeStruct((N, D), jnp.float32),
                   jax.ShapeDtypeStruct((N, D), jnp.float32)],
    )(nf, w1a, w1b, b1)


def _gather_add(p, q, senders, receivers):
    epw = E // NW
    nwin = epw // GW

    @functools.partial(
        pl.kernel,
        mesh=_mesh(),
        out_type=jax.ShapeDtypeStruct((E, D), jnp.float32),
        scratch_types=[
            pltpu.VMEM((epw,), jnp.int32),
            pltpu.VMEM((epw,), jnp.int32),
            pltpu.VMEM((GW, D), jnp.float32),
            pltpu.VMEM((GW, D), jnp.float32),
            pltpu.SemaphoreType.DMA,
            pltpu.SemaphoreType.DMA,
        ],
    )
    def k(p_hbm, q_hbm, s_hbm, r_hbm, g_hbm, sidx_v, ridx_v, vi_v, vj_v,
          sem_a, sem_b):
        wid = lax.axis_index("s") * NC + lax.axis_index("c")
        base = wid * epw
        ca = pltpu.async_copy(s_hbm.at[pl.ds(base, epw)], sidx_v, sem_a)
        cb = pltpu.async_copy(r_hbm.at[pl.ds(base, epw)], ridx_v, sem_b)
        ca.wait()
        cb.wait()

        @pl.loop(0, nwin)
        def _win(w):
            off = w * GW
            ga = pltpu.async_copy(p_hbm.at[sidx_v.at[pl.ds(off, GW)]], vi_v,
                                  sem_a)
            gb = pltpu.async_copy(q_hbm.at[ridx_v.at[pl.ds(off, GW)]], vj_v,
                                  sem_b)
            ga.wait()
            gb.wait()

            @pl.loop(0, GW)
            def _row(i):
                for c in range(D // LANES):
                    sl = pl.ds(c * LANES, LANES)
                    vi_v[i, sl] = vi_v[i, sl] + vj_v[i, sl]

            pltpu.sync_copy(vi_v, g_hbm.at[pl.ds(base + off, GW)])

    return k(p, q, senders, receivers)


def _edge_body(g, ef, w1c, w2, b2, w3, b3, gam, bet, ne, neo):
    efb = ef[...]
    h = g[...] + jnp.dot(efb, w1c[...], preferred_element_type=jnp.float32)
    h = jnp.maximum(h, 0.0)
    h = jnp.dot(h, w2[...], preferred_element_type=jnp.float32) + b2[...]
    h = jnp.maximum(h, 0.0)
    h = jnp.dot(h, w3[...], preferred_element_type=jnp.float32) + b3[...]
    mu = jnp.mean(h, axis=-1, keepdims=True)
    hc = h - mu
    var = jnp.mean(hc * hc, axis=-1, keepdims=True)
    ln = gam[...] * hc / jnp.sqrt(var + 1e-5) + bet[...]
    ne[...] = ln
    neo[...] = ln + efb


def _edge_mlp(g, ef, w1c, w2, b2, w3, b3, gam, bet):
    blk = 2000
    wspec = pl.BlockSpec((D, D), lambda i: (0, 0))
    bspec = pl.BlockSpec((1, D), lambda i: (0, 0))
    rspec = pl.BlockSpec((blk, D), lambda i: (i, 0))
    return pl.pallas_call(
        _edge_body,
        grid=(E // blk,),
        in_specs=[rspec, rspec, wspec, wspec, bspec, wspec, bspec, bspec,
                  bspec],
        out_specs=[rspec, rspec],
        out_shape=[jax.ShapeDtypeStruct((E, D), jnp.float32),
                   jax.ShapeDtypeStruct((E, D), jnp.float32)],
    )(g, ef, w1c, w2, b2, w3, b3, gam, bet)


def _scatter_add(new_edge, receivers, zeros):
    epc = E // NC
    epw = epc // NS
    nwin = epw // GW
    npw = 640

    @functools.partial(
        pl.kernel,
        mesh=_mesh(),
        out_type=jax.ShapeDtypeStruct((NC, N, D), jnp.float32),
        scratch_types=[
            pltpu.VMEM((1, GW), jnp.int32),
            pltpu.VMEM((GW, D), jnp.float32),
            pltpu.VMEM_SHARED((N, D), jnp.float32),
            pltpu.SemaphoreType.DMA,
        ],
    )
    def k(e_hbm, r_hbm, z_hbm, out_hbm, idx_v, rows_v, acc, sem):
        c = lax.axis_index("c")
        s = lax.axis_index("s")
        row0 = jnp.minimum(s * npw, N - npw)
        pltpu.sync_copy(z_hbm.at[pl.ds(row0, npw)],
                        acc.at[pl.ds(row0, npw)])
        plsc.subcore_barrier()

        base = c * epc + s * epw

        @pl.loop(0, nwin)
        def _win(w):
            off = base + w * GW
            pltpu.sync_copy(r_hbm.at[pl.ds(off, GW)], idx_v.at[0])
            pltpu.sync_copy(e_hbm.at[pl.ds(off, GW)], rows_v)
            pltpu.sync_copy(rows_v, acc.at[idx_v.at[0]], add=True)

        plsc.subcore_barrier()
        pltpu.sync_copy(acc.at[pl.ds(row0, npw)],
                        out_hbm.at[c].at[pl.ds(row0, npw)])

    return k(new_edge, receivers, zeros)


def _node_body(nf, parts, w1a, w1b, b1, w2, b2, w3, b3, gam, bet, out):
    nfb = nf[...]
    s = parts[0] + parts[1]
    h = (jnp.dot(nfb, w1a[...], preferred_element_type=jnp.float32)
         + jnp.dot(s, w1b[...], preferred_element_type=jnp.float32)
         + b1[...])
    h = jnp.maximum(h, 0.0)
    h = jnp.dot(h, w2[...], preferred_element_type=jnp.float32) + b2[...]
    h = jnp.maximum(h, 0.0)
    h = jnp.dot(h, w3[...], preferred_element_type=jnp.float32) + b3[...]
    mu = jnp.mean(h, axis=-1, keepdims=True)
    hc = h - mu
    var = jnp.mean(hc * hc, axis=-1, keepdims=True)
    ln = gam[...] * hc / jnp.sqrt(var + 1e-5) + bet[...]
    out[...] = ln + nfb


def _node_mlp(nf, parts, w1a, w1b, b1, w2, b2, w3, b3, gam, bet):
    blk = 2000
    wspec = pl.BlockSpec((D, D), lambda i: (0, 0))
    bspec = pl.BlockSpec((1, D), lambda i: (0, 0))
    rspec = pl.BlockSpec((blk, D), lambda i: (i, 0))
    pspec = pl.BlockSpec((NC, blk, D), lambda i: (0, i, 0))
    return pl.pallas_call(
        _node_body,
        grid=(N // blk,),
        in_specs=[rspec, pspec, wspec, wspec, bspec, wspec, bspec, wspec,
                  bspec, bspec, bspec],
        out_specs=pl.BlockSpec((blk, D), lambda i: (i, 0)),
        out_shape=jax.ShapeDtypeStruct((N, D), jnp.float32),
    )(nf, parts, w1a, w1b, b1, w2, b2, w3, b3, gam, bet)


def kernel(node_features, edge_features, senders, receivers,
           eW1, eb1, eW2, eb2, eW3, eb3, eg, ebt,
           nW1, nb1, nW2, nb2, nW3, nb3, ng, nbt):
    nf = node_features.reshape(N, D)
    ef = edge_features.reshape(E, D)

    p, q = _compute_pq(nf, eW1[:D], eW1[D:2 * D], eb1.reshape(1, D))
    g = _gather_add(p, q, senders, receivers)
    ne, neo = _edge_mlp(g, ef, eW1[2 * D:], eW2, eb2.reshape(1, D),
                        eW3, eb3.reshape(1, D), eg.reshape(1, D),
                        ebt.reshape(1, D))
    zeros = jnp.zeros((N, D), jnp.float32)
    parts = _scatter_add(ne, receivers, zeros)
    nn = _node_mlp(nf, parts, nW1[:D], nW1[D:2 * D], nb1.reshape(1, D),
                   nW2, nb2.reshape(1, D), nW3, nb3.reshape(1, D),
                   ng.reshape(1, D), nbt.reshape(1, D))
    return nn.reshape(1, N, D), neo.reshape(1, E, D)


# speedup vs baseline: 5.1531x; 1.4588x over previous
import functools

import jax
import jax.numpy as jnp
from jax import lax
from jax.experimental import pallas as pl
from jax.experimental.pallas import tpu as pltpu
from jax.experimental.pallas import tpu_sc as plsc

N = 10000
E = 320000
D = 128

NC = 2
NS = 16
NW = NC * NS
LANES = 16

GW = 80


def _mesh():
    return plsc.VectorSubcoreMesh(core_axis_name="c", subcore_axis_name="s")


def _pq_body(nf, w1a, w1b, b1, p_out, q_out):
    x = nf[...]
    p_out[...] = jnp.dot(x, w1a[...], preferred_element_type=jnp.float32) + b1[...]
    q_out[...] = jnp.dot(x, w1b[...], preferred_element_type=jnp.float32)


def _compute_pq(nf, w1a, w1b, b1):
    blk = 2000
    return pl.pallas_call(
        _pq_body,
        grid=(N // blk,),
        in_specs=[
            pl.BlockSpec((blk, D), lambda i: (i, 0)),
            pl.BlockSpec((D, D), lambda i: (0, 0)),
            pl.BlockSpec((D, D), lambda i: (0, 0)),
            pl.BlockSpec((1, D), lambda i: (0, 0)),
        ],
        out_specs=[pl.BlockSpec((blk, D), lambda i: (i, 0)),
                   pl.BlockSpec((blk, D), lambda i: (i, 0))],
        out_shape=[jax.ShapeDtypeStruct((N, D), jnp.float32),
                   jax.ShapeDtypeStruct((N, D), jnp.float32)],
    )(nf, w1a, w1b, b1)


def _gather_add(p, q, senders, receivers):
    epw = E // NW
    nwin = epw // GW

    @functools.partial(
        pl.kernel,
        mesh=_mesh(),
        out_type=jax.ShapeDtypeStruct((E, D), jnp.float32),
        scratch_types=[
            pltpu.VMEM((epw,), jnp.int32),
            pltpu.VMEM((epw,), jnp.int32),
            pltpu.VMEM((2, GW, D), jnp.float32),
            pltpu.VMEM((2, GW, D), jnp.float32),
            pltpu.VMEM((2, GW, D), jnp.float32),
            pltpu.SemaphoreType.DMA,
            pltpu.SemaphoreType.DMA,
            pltpu.SemaphoreType.DMA,
            pltpu.SemaphoreType.DMA,
        ],
    )
    def k(p_hbm, q_hbm, s_hbm, r_hbm, g_hbm, sidx_v, ridx_v, vi, vj, go,
          gs0, gs1, ws0, ws1):
        wid = lax.axis_index("s") * NC + lax.axis_index("c")
        base = wid * epw
        gsems = (gs0, gs1)
        wsems = (ws0, ws1)
        pltpu.async_copy(s_hbm.at[pl.ds(base, epw)], sidx_v, gs0)
        pltpu.async_copy(r_hbm.at[pl.ds(base, epw)], ridx_v, gs1)
        pltpu.make_async_copy(s_hbm.at[pl.ds(base, epw)], sidx_v, gs0).wait()
        pltpu.make_async_copy(r_hbm.at[pl.ds(base, epw)], ridx_v, gs1).wait()

        def issue(w, b):
            off = w * GW
            pltpu.async_copy(p_hbm.at[sidx_v.at[pl.ds(off, GW)]], vi.at[b],
                             gsems[b])
            pltpu.async_copy(q_hbm.at[ridx_v.at[pl.ds(off, GW)]], vj.at[b],
                             gsems[b])

        def wait_gather(b):
            pltpu.make_async_copy(p_hbm.at[pl.ds(0, GW)], vi.at[b],
                                  gsems[b]).wait()
            pltpu.make_async_copy(p_hbm.at[pl.ds(0, GW)], vj.at[b],
                                  gsems[b]).wait()

        def wait_write(b):
            pltpu.make_async_copy(p_hbm.at[pl.ds(0, GW)], go.at[b],
                                  wsems[b]).wait()

        def vadd(b):
            @pl.loop(0, GW)
            def _row(i):
                for c in range(D // LANES):
                    sl = pl.ds(c * LANES, LANES)
                    go[b, i, sl] = vi[b, i, sl] + vj[b, i, sl]

        def write(w, b):
            pltpu.async_copy(go.at[b], g_hbm.at[pl.ds(base + w * GW, GW)],
                             wsems[b])

        issue(0, 0)
        issue(1, 1)
        for b in (0, 1):
            wait_gather(b)
            vadd(b)
            issue(b + 2, b)
            write(b, b)

        @pl.loop(1, (nwin - 3) // 2)
        def _main(i):
            for b in (0, 1):
                w = 2 * i + b
                wait_gather(b)
                wait_write(b)
                vadd(b)
                issue(w + 2, b)
                write(w, b)

        wait_gather(0)
        wait_write(0)
        vadd(0)
        issue(nwin - 1, 0)
        write(nwin - 3, 0)

        wait_gather(1)
        wait_write(1)
        vadd(1)
        write(nwin - 2, 1)

        wait_gather(0)
        wait_write(0)
        vadd(0)
        write(nwin - 1, 0)

        wait_write(0)
        wait_write(1)

    return k(p, q, senders, receivers)


def _edge_body(g, ef, w1c, w2, b2, w3, b3, gam, bet, ne, neo):
    efb = ef[...]
    h = g[...] + jnp.dot(efb, w1c[...], preferred_element_type=jnp.float32)
    h = jnp.maximum(h, 0.0)
    h = jnp.dot(h, w2[...], preferred_element_type=jnp.float32) + b2[...]
    h = jnp.maximum(h, 0.0)
    h = jnp.dot(h, w3[...], preferred_element_type=jnp.float32) + b3[...]
    mu = jnp.mean(h, axis=-1, keepdims=True)
    hc = h - mu
    var = jnp.mean(hc * hc, axis=-1, keepdims=True)
    ln = gam[...] * hc / jnp.sqrt(var + 1e-5) + bet[...]
    ne[...] = ln
    neo[...] = ln + efb


def _edge_mlp(g, ef, w1c, w2, b2, w3, b3, gam, bet):
    blk = 2000
    wspec = pl.BlockSpec((D, D), lambda i: (0, 0))
    bspec = pl.BlockSpec((1, D), lambda i: (0, 0))
    rspec = pl.BlockSpec((blk, D), lambda i: (i, 0))
    return pl.pallas_call(
        _edge_body,
        grid=(E // blk,),
        in_specs=[rspec, rspec, wspec, wspec, bspec, wspec, bspec, bspec,
                  bspec],
        out_specs=[rspec, rspec],
        out_shape=[jax.ShapeDtypeStruct((E, D), jnp.float32),
                   jax.ShapeDtypeStruct((E, D), jnp.float32)],
    )(g, ef, w1c, w2, b2, w3, b3, gam, bet)


NSLOT = 4


def _scatter_add(new_edge, receivers, zeros):
    epc = E // NC
    epw = epc // NS
    nwin = epw // GW
    npw = 640

    @functools.partial(
        pl.kernel,
        mesh=_mesh(),
        out_type=jax.ShapeDtypeStruct((NC, N, D), jnp.float32),
        scratch_types=[
            pltpu.VMEM((NSLOT, GW), jnp.int32),
            pltpu.VMEM((NSLOT, GW, D), jnp.float32),
            pltpu.VMEM_SHARED((N, D), jnp.float32),
        ] + [pltpu.SemaphoreType.DMA] * (2 * NSLOT),
    )
    def k(e_hbm, r_hbm, z_hbm, out_hbm, idx_v, rows_v, acc, *sems):
        lsems = sems[:NSLOT]
        ssems = sems[NSLOT:]
        c = lax.axis_index("c")
        s = lax.axis_index("s")
        base = c * epc + s * epw

        def issue_load(w, b):
            off = base + w * GW
            pltpu.async_copy(r_hbm.at[pl.ds(off, GW)], idx_v.at[b], lsems[b])
            pltpu.async_copy(e_hbm.at[pl.ds(off, GW)], rows_v.at[b], lsems[b])

        def wait_load(b):
            pltpu.make_async_copy(r_hbm.at[pl.ds(0, GW)], idx_v.at[b],
                                  lsems[b]).wait()
            pltpu.make_async_copy(e_hbm.at[pl.ds(0, GW)], rows_v.at[b],
                                  lsems[b]).wait()

        def scat(b):
            pltpu.async_copy(rows_v.at[b], acc.at[idx_v.at[b]], ssems[b],
                             add=True)

        def wait_scat(b):
            pltpu.make_async_copy(e_hbm.at[pl.ds(0, GW)], rows_v.at[b],
                                  ssems[b]).wait()

        issue_load(0, 0)
        issue_load(1, 1)
        row0 = jnp.minimum(s * npw, N - npw)
        pltpu.sync_copy(z_hbm.at[pl.ds(row0, npw)],
                        acc.at[pl.ds(row0, npw)])
        plsc.subcore_barrier()

        wait_load(0); scat(0); issue_load(2, 2)
        wait_load(1); scat(1); issue_load(3, 3)
        wait_load(2); scat(2); wait_scat(0); issue_load(4, 0)
        wait_load(3); scat(3); wait_scat(1); issue_load(5, 1)

        @pl.loop(1, (nwin - 5) // NSLOT)
        def _main(i):
            for j in range(NSLOT):
                w = NSLOT * i + j
                wait_load(j)
                scat(j)
                b2 = (j + 2) % NSLOT
                wait_scat(b2)
                issue_load(w + 2, b2)

        wait_load(0); scat(0); wait_scat(2); issue_load(nwin - 3, 2)
        wait_load(1); scat(1); wait_scat(3); issue_load(nwin - 2, 3)
        wait_load(2); scat(2); wait_scat(0); issue_load(nwin - 1, 0)
        wait_load(3); scat(3)
        wait_load(0); scat(0)
        wait_scat(1)
        wait_scat(2)
        wait_scat(3)
        wait_scat(0)

        plsc.subcore_barrier()
        pltpu.sync_copy(acc.at[pl.ds(row0, npw)],
                        out_hbm.at[c].at[pl.ds(row0, npw)])

    return k(new_edge, receivers, zeros)


def _node_body(nf, parts, w1a, w1b, b1, w2, b2, w3, b3, gam, bet, out):
    nfb = nf[...]
    s = parts[0] + parts[1]
    h = (jnp.dot(nfb, w1a[...], preferred_element_type=jnp.float32)
         + jnp.dot(s, w1b[...], preferred_element_type=jnp.float32)
         + b1[...])
    h = jnp.maximum(h, 0.0)
    h = jnp.dot(h, w2[...], preferred_element_type=jnp.float32) + b2[...]
    h = jnp.maximum(h, 0.0)
    h = jnp.dot(h, w3[...], preferred_element_type=jnp.float32) + b3[...]
    mu = jnp.mean(h, axis=-1, keepdims=True)
    hc = h - mu
    var = jnp.mean(hc * hc, axis=-1, keepdims=True)
    ln = gam[...] * hc / jnp.sqrt(var + 1e-5) + bet[...]
    out[...] = ln + nfb


def _node_mlp(nf, parts, w1a, w1b, b1, w2, b2, w3, b3, gam, bet):
    blk = 2000
    wspec = pl.BlockSpec((D, D), lambda i: (0, 0))
    bspec = pl.BlockSpec((1, D), lambda i: (0, 0))
    rspec = pl.BlockSpec((blk, D), lambda i: (i, 0))
    pspec = pl.BlockSpec((NC, blk, D), lambda i: (0, i, 0))
    return pl.pallas_call(
        _node_body,
        grid=(N // blk,),
        in_specs=[rspec, pspec, wspec, wspec, bspec, wspec, bspec, wspec,
                  bspec, bspec, bspec],
        out_specs=pl.BlockSpec((blk, D), lambda i: (i, 0)),
        out_shape=jax.ShapeDtypeStruct((N, D), jnp.float32),
    )(nf, parts, w1a, w1b, b1, w2, b2, w3, b3, gam, bet)


def kernel(node_features, edge_features, senders, receivers,
           eW1, eb1, eW2, eb2, eW3, eb3, eg, ebt,
           nW1, nb1, nW2, nb2, nW3, nb3, ng, nbt):
    nf = node_features.reshape(N, D)
    ef = edge_features.reshape(E, D)

    p, q = _compute_pq(nf, eW1[:D], eW1[D:2 * D], eb1.reshape(1, D))
    g = _gather_add(p, q, senders, receivers)
    ne, neo = _edge_mlp(g, ef, eW1[2 * D:], eW2, eb2.reshape(1, D),
                        eW3, eb3.reshape(1, D), eg.reshape(1, D),
                        ebt.reshape(1, D))
    zeros = jnp.zeros((N, D), jnp.float32)
    parts = _scatter_add(ne, receivers, zeros)
    nn = _node_mlp(nf, parts, nW1[:D], nW1[D:2 * D], nb1.reshape(1, D),
                   nW2, nb2.reshape(1, D), nW3, nb3.reshape(1, D),
                   ng.reshape(1, D), nbt.reshape(1, D))
    return nn.reshape(1, N, D), neo.reshape(1, E, D)


# speedup vs baseline: 5.2293x; 1.0148x over previous
import functools

import jax
import jax.numpy as jnp
from jax import lax
from jax.experimental import pallas as pl
from jax.experimental.pallas import tpu as pltpu
from jax.experimental.pallas import tpu_sc as plsc

N = 10000
E = 320000
D = 128

NC = 2
NS = 16
NW = NC * NS
LANES = 16

GW = 80


def _mesh():
    return plsc.VectorSubcoreMesh(core_axis_name="c", subcore_axis_name="s")


def _pq_body(nf, w1a, w1b, b1, p_out, q_out):
    x = nf[...]
    p_out[...] = jnp.dot(x, w1a[...], preferred_element_type=jnp.float32) + b1[...]
    q_out[...] = jnp.dot(x, w1b[...], preferred_element_type=jnp.float32)


def _compute_pq(nf, w1a, w1b, b1):
    blk = 2000
    return pl.pallas_call(
        _pq_body,
        grid=(N // blk,),
        in_specs=[
            pl.BlockSpec((blk, D), lambda i: (i, 0)),
            pl.BlockSpec((D, D), lambda i: (0, 0)),
            pl.BlockSpec((D, D), lambda i: (0, 0)),
            pl.BlockSpec((1, D), lambda i: (0, 0)),
        ],
        out_specs=[pl.BlockSpec((blk, D), lambda i: (i, 0)),
                   pl.BlockSpec((blk, D), lambda i: (i, 0))],
        out_shape=[jax.ShapeDtypeStruct((N, D), jnp.float32),
                   jax.ShapeDtypeStruct((N, D), jnp.float32)],
    )(nf, w1a, w1b, b1)


def _gather_add(p, q, senders, receivers):
    epw = E // NW
    nwin = epw // GW

    @functools.partial(
        pl.kernel,
        mesh=_mesh(),
        out_type=jax.ShapeDtypeStruct((E, D), jnp.float32),
        scratch_types=[
            pltpu.VMEM((epw,), jnp.int32),
            pltpu.VMEM((epw,), jnp.int32),
            pltpu.VMEM((3, GW, D), jnp.float32),
            pltpu.VMEM((3, GW, D), jnp.float32),
            pltpu.VMEM((3, GW, D), jnp.float32),
        ] + [pltpu.SemaphoreType.DMA] * 6,
    )
    def k(p_hbm, q_hbm, s_hbm, r_hbm, g_hbm, sidx_v, ridx_v, vi, vj, go,
          *sems):
        gsems = sems[:3]
        wsems = sems[3:]
        wid = lax.axis_index("s") * NC + lax.axis_index("c")
        base = wid * epw
        pltpu.async_copy(s_hbm.at[pl.ds(base, epw)], sidx_v, gsems[0])
        pltpu.async_copy(r_hbm.at[pl.ds(base, epw)], ridx_v, gsems[1])
        pltpu.make_async_copy(s_hbm.at[pl.ds(base, epw)], sidx_v,
                              gsems[0]).wait()
        pltpu.make_async_copy(r_hbm.at[pl.ds(base, epw)], ridx_v,
                              gsems[1]).wait()

        def issue(w, b):
            off = w * GW
            pltpu.async_copy(p_hbm.at[sidx_v.at[pl.ds(off, GW)]], vi.at[b],
                             gsems[b])
            pltpu.async_copy(q_hbm.at[ridx_v.at[pl.ds(off, GW)]], vj.at[b],
                             gsems[b])

        def wait_gather(b):
            pltpu.make_async_copy(p_hbm.at[pl.ds(0, GW)], vi.at[b],
                                  gsems[b]).wait()
            pltpu.make_async_copy(p_hbm.at[pl.ds(0, GW)], vj.at[b],
                                  gsems[b]).wait()

        def wait_write(b):
            pltpu.make_async_copy(p_hbm.at[pl.ds(0, GW)], go.at[b],
                                  wsems[b]).wait()

        def vadd(b):
            @plsc.parallel_loop(0, GW, unroll=4)
            def _row(i):
                for c in range(D // LANES):
                    sl = pl.ds(c * LANES, LANES)
                    go[b, i, sl] = vi[b, i, sl] + vj[b, i, sl]

        def write(w, b):
            pltpu.async_copy(go.at[b], g_hbm.at[pl.ds(base + w * GW, GW)],
                             wsems[b])

        issue(0, 0)
        issue(1, 1)
        issue(2, 2)
        for b in (0, 1, 2):
            wait_gather(b)
            vadd(b)
            issue(b + 3, b)
            write(b, b)

        @pl.loop(1, (nwin - 5) // 3)
        def _main(i):
            for b in (0, 1, 2):
                w = 3 * i + b
                wait_gather(b)
                wait_write(b)
                vadd(b)
                issue(w + 3, b)
                write(w, b)

        wait_gather(0)
        wait_write(0)
        vadd(0)
        issue(nwin - 2, 0)
        write(nwin - 5, 0)

        wait_gather(1)
        wait_write(1)
        vadd(1)
        issue(nwin - 1, 1)
        write(nwin - 4, 1)

        for b, w in ((2, nwin - 3), (0, nwin - 2), (1, nwin - 1)):
            wait_gather(b)
            wait_write(b)
            vadd(b)
            write(w, b)

        wait_write(2)
        wait_write(0)
        wait_write(1)

    return k(p, q, senders, receivers)


def _edge_body(g, ef, w1c, w2, b2, w3, b3, gam, bet, ne, neo):
    efb = ef[...]
    h = g[...] + jnp.dot(efb, w1c[...], preferred_element_type=jnp.float32)
    h = jnp.maximum(h, 0.0)
    h = jnp.dot(h, w2[...], preferred_element_type=jnp.float32) + b2[...]
    h = jnp.maximum(h, 0.0)
    h = jnp.dot(h, w3[...], preferred_element_type=jnp.float32) + b3[...]
    mu = jnp.mean(h, axis=-1, keepdims=True)
    hc = h - mu
    var = jnp.mean(hc * hc, axis=-1, keepdims=True)
    ln = gam[...] * hc / jnp.sqrt(var + 1e-5) + bet[...]
    ne[...] = ln
    neo[...] = ln + efb


def _edge_mlp(g, ef, w1c, w2, b2, w3, b3, gam, bet):
    blk = 2000
    wspec = pl.BlockSpec((D, D), lambda i: (0, 0))
    bspec = pl.BlockSpec((1, D), lambda i: (0, 0))
    rspec = pl.BlockSpec((blk, D), lambda i: (i, 0))
    return pl.pallas_call(
        _edge_body,
        grid=(E // blk,),
        in_specs=[rspec, rspec, wspec, wspec, bspec, wspec, bspec, bspec,
                  bspec],
        out_specs=[rspec, rspec],
        out_shape=[jax.ShapeDtypeStruct((E, D), jnp.float32),
                   jax.ShapeDtypeStruct((E, D), jnp.float32)],
    )(g, ef, w1c, w2, b2, w3, b3, gam, bet)


NSLOT = 4


def _scatter_add(new_edge, receivers, zeros):
    epc = E // NC
    epw = epc // NS
    nwin = epw // GW
    npw = 640

    @functools.partial(
        pl.kernel,
        mesh=_mesh(),
        out_type=jax.ShapeDtypeStruct((NC, N, D), jnp.float32),
        scratch_types=[
            pltpu.VMEM((NSLOT, GW), jnp.int32),
            pltpu.VMEM((NSLOT, GW, D), jnp.float32),
            pltpu.VMEM_SHARED((N, D), jnp.float32),
        ] + [pltpu.SemaphoreType.DMA] * (2 * NSLOT),
    )
    def k(e_hbm, r_hbm, z_hbm, out_hbm, idx_v, rows_v, acc, *sems):
        lsems = sems[:NSLOT]
        ssems = sems[NSLOT:]
        c = lax.axis_index("c")
        s = lax.axis_index("s")
        base = c * epc + s * epw

        def issue_load(w, b):
            off = base + w * GW
            pltpu.async_copy(r_hbm.at[pl.ds(off, GW)], idx_v.at[b], lsems[b])
            pltpu.async_copy(e_hbm.at[pl.ds(off, GW)], rows_v.at[b], lsems[b])

        def wait_load(b):
            pltpu.make_async_copy(r_hbm.at[pl.ds(0, GW)], idx_v.at[b],
                                  lsems[b]).wait()
            pltpu.make_async_copy(e_hbm.at[pl.ds(0, GW)], rows_v.at[b],
                                  lsems[b]).wait()

        def scat(b):
            pltpu.async_copy(rows_v.at[b], acc.at[idx_v.at[b]], ssems[b],
                             add=True)

        def wait_scat(b):
            pltpu.make_async_copy(e_hbm.at[pl.ds(0, GW)], rows_v.at[b],
                                  ssems[b]).wait()

        issue_load(0, 0)
        issue_load(1, 1)
        row0 = jnp.minimum(s * npw, N - npw)
        pltpu.sync_copy(z_hbm.at[pl.ds(row0, npw)],
                        acc.at[pl.ds(row0, npw)])
        plsc.subcore_barrier()

        wait_load(0); scat(0); issue_load(2, 2)
        wait_load(1); scat(1); issue_load(3, 3)
        wait_load(2); scat(2); wait_scat(0); issue_load(4, 0)
        wait_load(3); scat(3); wait_scat(1); issue_load(5, 1)

        @pl.loop(1, (nwin - 5) // NSLOT)
        def _main(i):
            for j in range(NSLOT):
                w = NSLOT * i + j
                wait_load(j)
                scat(j)
                b2 = (j + 2) % NSLOT
                wait_scat(b2)
                issue_load(w + 2, b2)

        wait_load(0); scat(0); wait_scat(2); issue_load(nwin - 3, 2)
        wait_load(1); scat(1); wait_scat(3); issue_load(nwin - 2, 3)
        wait_load(2); scat(2); wait_scat(0); issue_load(nwin - 1, 0)
        wait_load(3); scat(3)
        wait_load(0); scat(0)
        wait_scat(1)
        wait_scat(2)
        wait_scat(3)
        wait_scat(0)

        plsc.subcore_barrier()
        pltpu.sync_copy(acc.at[pl.ds(row0, npw)],
                        out_hbm.at[c].at[pl.ds(row0, npw)])

    return k(new_edge, receivers, zeros)


def _node_body(nf, parts, w1a, w1b, b1, w2, b2, w3, b3, gam, bet, out):
    nfb = nf[...]
    s = parts[0] + parts[1]
    h = (jnp.dot(nfb, w1a[...], preferred_element_type=jnp.float32)
         + jnp.dot(s, w1b[...], preferred_element_type=jnp.float32)
         + b1[...])
    h = jnp.maximum(h, 0.0)
    h = jnp.dot(h, w2[...], preferred_element_type=jnp.float32) + b2[...]
    h = jnp.maximum(h, 0.0)
    h = jnp.dot(h, w3[...], preferred_element_type=jnp.float32) + b3[...]
    mu = jnp.mean(h, axis=-1, keepdims=True)
    hc = h - mu
    var = jnp.mean(hc * hc, axis=-1, keepdims=True)
    ln = gam[...] * hc / jnp.sqrt(var + 1e-5) + bet[...]
    out[...] = ln + nfb


def _node_mlp(nf, parts, w1a, w1b, b1, w2, b2, w3, b3, gam, bet):
    blk = 2000
    wspec = pl.BlockSpec((D, D), lambda i: (0, 0))
    bspec = pl.BlockSpec((1, D), lambda i: (0, 0))
    rspec = pl.BlockSpec((blk, D), lambda i: (i, 0))
    pspec = pl.BlockSpec((NC, blk, D), lambda i: (0, i, 0))
    return pl.pallas_call(
        _node_body,
        grid=(N // blk,),
        in_specs=[rspec, pspec, wspec, wspec, bspec, wspec, bspec, wspec,
                  bspec, bspec, bspec],
        out_specs=pl.BlockSpec((blk, D), lambda i: (i, 0)),
        out_shape=jax.ShapeDtypeStruct((N, D), jnp.float32),
    )(nf, parts, w1a, w1b, b1, w2, b2, w3, b3, gam, bet)


def kernel(node_features, edge_features, senders, receivers,
           eW1, eb1, eW2, eb2, eW3, eb3, eg, ebt,
           nW1, nb1, nW2, nb2, nW3, nb3, ng, nbt):
    nf = node_features.reshape(N, D)
    ef = edge_features.reshape(E, D)

    p, q = _compute_pq(nf, eW1[:D], eW1[D:2 * D], eb1.reshape(1, D))
    g = _gather_add(p, q, senders, receivers)
    ne, neo = _edge_mlp(g, ef, eW1[2 * D:], eW2, eb2.reshape(1, D),
                        eW3, eb3.reshape(1, D), eg.reshape(1, D),
                        ebt.reshape(1, D))
    zeros = jnp.zeros((N, D), jnp.float32)
    parts = _scatter_add(ne, receivers, zeros)
    nn = _node_mlp(nf, parts, nW1[:D], nW1[D:2 * D], nb1.reshape(1, D),
                   nW2, nb2.reshape(1, D), nW3, nb3.reshape(1, D),
                   ng.reshape(1, D), nbt.reshape(1, D))
    return nn.reshape(1, N, D), neo.reshape(1, E, D)


# speedup vs baseline: 5.6996x; 1.0899x over previous
import functools

import jax
import jax.numpy as jnp
from jax import lax
from jax.experimental import pallas as pl
from jax.experimental.pallas import tpu as pltpu
from jax.experimental.pallas import tpu_sc as plsc

N = 10000
E = 320000
D = 128

NC = 2
NS = 16
NW = NC * NS
LANES = 16

GW = 40
NCHUNK = 2
CHUNK = E // NCHUNK
EBLK = 2000
NBLK = 2000


def _mesh():
    return plsc.VectorSubcoreMesh(core_axis_name="c", subcore_axis_name="s")


def _pq_body(nf, w1a, w1b, b1, p_out, q_out):
    x = nf[...]
    p_out[...] = jnp.dot(x, w1a[...], preferred_element_type=jnp.float32) + b1[...]
    q_out[...] = jnp.dot(x, w1b[...], preferred_element_type=jnp.float32)


def _compute_pq(nf, w1a, w1b, b1):
    blk = 2000
    return pl.pallas_call(
        _pq_body,
        grid=(N // blk,),
        in_specs=[
            pl.BlockSpec((blk, D), lambda i: (i, 0)),
            pl.BlockSpec((D, D), lambda i: (0, 0)),
            pl.BlockSpec((D, D), lambda i: (0, 0)),
            pl.BlockSpec((1, D), lambda i: (0, 0)),
        ],
        out_specs=[pl.BlockSpec((blk, D), lambda i: (i, 0)),
                   pl.BlockSpec((blk, D), lambda i: (i, 0))],
        out_shape=[jax.ShapeDtypeStruct((N, D), jnp.float32),
                   jax.ShapeDtypeStruct((N, D), jnp.float32)],
    )(nf, w1a, w1b, b1)


def _gather_add(p, q, senders, receivers, chunk):
    epw = CHUNK // NW
    nwin = epw // GW
    assert epw % GW == 0 and nwin >= 8

    @functools.partial(
        pl.kernel,
        mesh=_mesh(),
        out_type=jax.ShapeDtypeStruct((CHUNK, D), jnp.float32),
        scratch_types=[
            pltpu.VMEM((epw,), jnp.int32),
            pltpu.VMEM((epw,), jnp.int32),
            pltpu.VMEM((3, GW, D), jnp.float32),
            pltpu.VMEM((3, GW, D), jnp.float32),
            pltpu.VMEM((3, GW, D), jnp.float32),
        ] + [pltpu.SemaphoreType.DMA] * 6,
    )
    def k(p_hbm, q_hbm, s_hbm, r_hbm, g_hbm, sidx_v, ridx_v, vi, vj, go,
          *sems):
        gsems = sems[:3]
        wsems = sems[3:]
        wid = lax.axis_index("s") * NC + lax.axis_index("c")
        base = wid * epw
        gbase = chunk * CHUNK + base
        pltpu.async_copy(s_hbm.at[pl.ds(gbase, epw)], sidx_v, gsems[0])
        pltpu.async_copy(r_hbm.at[pl.ds(gbase, epw)], ridx_v, gsems[1])
        pltpu.make_async_copy(s_hbm.at[pl.ds(gbase, epw)], sidx_v,
                              gsems[0]).wait()
        pltpu.make_async_copy(r_hbm.at[pl.ds(gbase, epw)], ridx_v,
                              gsems[1]).wait()

        def issue(w, b):
            off = w * GW
            pltpu.async_copy(p_hbm.at[sidx_v.at[pl.ds(off, GW)]], vi.at[b],
                             gsems[b])
            pltpu.async_copy(q_hbm.at[ridx_v.at[pl.ds(off, GW)]], vj.at[b],
                             gsems[b])

        def wait_gather(b):
            pltpu.make_async_copy(p_hbm.at[pl.ds(0, GW)], vi.at[b],
                                  gsems[b]).wait()
            pltpu.make_async_copy(p_hbm.at[pl.ds(0, GW)], vj.at[b],
                                  gsems[b]).wait()

        def wait_write(b):
            pltpu.make_async_copy(p_hbm.at[pl.ds(0, GW)], go.at[b],
                                  wsems[b]).wait()

        def vadd(b):
            @plsc.parallel_loop(0, GW, unroll=4)
            def _row(i):
                for c in range(D // LANES):
                    sl = pl.ds(c * LANES, LANES)
                    go[b, i, sl] = vi[b, i, sl] + vj[b, i, sl]

        def write(w, b):
            pltpu.async_copy(go.at[b], g_hbm.at[pl.ds(base + w * GW, GW)],
                             wsems[b])

        def step(w, b, first, last):
            wait_gather(b)
            if not first:
                wait_write(b)
            vadd(b)
            if not last:
                issue(w + 3, b)
            write(w, b)

        p_extra = (nwin - 6) % 3
        m = (nwin - 6) // 3
        issue(0, 0)
        issue(1, 1)
        issue(2, 2)
        for w in range(3):
            step(w, w, True, False)
        for w in range(3, 3 + p_extra):
            step(w, w % 3, False, False)

        w0 = 3 + p_extra
        slots = (w0 % 3, (w0 + 1) % 3, (w0 + 2) % 3)

        @pl.loop(0, m)
        def _main(i):
            for t in range(3):
                step(w0 + 3 * i + t, slots[t], False, False)

        for w in range(nwin - 3, nwin):
            step(w, w % 3, False, True)
        for w in range(nwin - 3, nwin):
            wait_write(w % 3)

    return k(p, q, senders, receivers)


def _edge_body_noprev(g, ef, w1c, w2, b2, w3, b3, gam, bet, neo, ne):
    efb = ef[...]
    h = g[...] + jnp.dot(efb, w1c[...], preferred_element_type=jnp.float32)
    h = jnp.maximum(h, 0.0)
    h = jnp.dot(h, w2[...], preferred_element_type=jnp.float32) + b2[...]
    h = jnp.maximum(h, 0.0)
    h = jnp.dot(h, w3[...], preferred_element_type=jnp.float32) + b3[...]
    mu = jnp.mean(h, axis=-1, keepdims=True)
    hc = h - mu
    var = jnp.mean(hc * hc, axis=-1, keepdims=True)
    ln = gam[...] * hc / jnp.sqrt(var + 1e-5) + bet[...]
    ne[...] = ln
    neo[...] = ln + efb


def _edge_body_prev(g, ef, w1c, w2, b2, w3, b3, gam, bet, prev, neo, ne):
    del prev
    _edge_body_noprev(g, ef, w1c, w2, b2, w3, b3, gam, bet, neo, ne)


def _edge_mlp(g, ef, w1c, w2, b2, w3, b3, gam, bet, chunk, neo_prev):
    nblk = CHUNK // EBLK
    off = chunk * nblk
    wspec = pl.BlockSpec((D, D), lambda i: (0, 0))
    bspec = pl.BlockSpec((1, D), lambda i: (0, 0))
    cspec = pl.BlockSpec((EBLK, D), lambda i: (i, 0))
    fspec = pl.BlockSpec((EBLK, D), lambda i: (i + off, 0))
    in_specs = [cspec, fspec, wspec, wspec, bspec, wspec, bspec, bspec,
                bspec]
    args = [g, ef, w1c, w2, b2, w3, b3, gam, bet]
    kwargs = {}
    if neo_prev is None:
        body = _edge_body_noprev
    else:
        body = _edge_body_prev
        in_specs = in_specs + [pl.BlockSpec(memory_space=pl.ANY)]
        args = args + [neo_prev]
        kwargs["input_output_aliases"] = {len(args) - 1: 0}
    return pl.pallas_call(
        body,
        grid=(nblk,),
        in_specs=in_specs,
        out_specs=[fspec, cspec],
        out_shape=[jax.ShapeDtypeStruct((E, D), jnp.float32),
                   jax.ShapeDtypeStruct((CHUNK, D), jnp.float32)],
        **kwargs,
    )(*args)


NSLOT = 4


def _scatter_add(new_edge, receivers, zeros, chunk):
    epc = CHUNK // NC
    epw = epc // NS
    nwin = epw // GW
    assert epw % GW == 0 and nwin >= 8
    npw = 640

    @functools.partial(
        pl.kernel,
        mesh=_mesh(),
        out_type=jax.ShapeDtypeStruct((NC, N, D), jnp.float32),
        scratch_types=[
            pltpu.VMEM((NSLOT, GW), jnp.int32),
            pltpu.VMEM((NSLOT, GW, D), jnp.float32),
            pltpu.VMEM_SHARED((N, D), jnp.float32),
        ] + [pltpu.SemaphoreType.DMA] * (2 * NSLOT),
    )
    def k(e_hbm, r_hbm, z_hbm, out_hbm, idx_v, rows_v, acc, *sems):
        lsems = sems[:NSLOT]
        ssems = sems[NSLOT:]
        c = lax.axis_index("c")
        s = lax.axis_index("s")
        base = c * epc + s * epw
        rbase = chunk * CHUNK + base

        def issue_load(w, b):
            off = w * GW
            pltpu.async_copy(r_hbm.at[pl.ds(rbase + off, GW)], idx_v.at[b],
                             lsems[b])
            pltpu.async_copy(e_hbm.at[pl.ds(base + off, GW)], rows_v.at[b],
                             lsems[b])

        def wait_load(b):
            pltpu.make_async_copy(r_hbm.at[pl.ds(0, GW)], idx_v.at[b],
                                  lsems[b]).wait()
            pltpu.make_async_copy(e_hbm.at[pl.ds(0, GW)], rows_v.at[b],
                                  lsems[b]).wait()

        def scat(b):
            pltpu.async_copy(rows_v.at[b], acc.at[idx_v.at[b]], ssems[b],
                             add=True)

        def wait_scat(b):
            pltpu.make_async_copy(e_hbm.at[pl.ds(0, GW)], rows_v.at[b],
                                  ssems[b]).wait()

        issue_load(0, 0)
        issue_load(1, 1)
        row0 = jnp.minimum(s * npw, N - npw)
        pltpu.sync_copy(z_hbm.at[pl.ds(row0, npw)],
                        acc.at[pl.ds(row0, npw)])
        plsc.subcore_barrier()

        def step(w, b, first, last):
            wait_load(b)
            scat(b)
            if not last:
                b2 = (b + 2) % NSLOT
                if not first:
                    wait_scat(b2)
                issue_load(w + 2, b2)

        p_extra = (nwin - 4) % NSLOT
        m = (nwin - 4) // NSLOT
        step(0, 0, True, False)
        step(1, 1, True, False)
        for w in range(2, 2 + p_extra):
            step(w, w % NSLOT, False, False)

        w0 = 2 + p_extra
        slots = tuple((w0 + t) % NSLOT for t in range(NSLOT))

        @pl.loop(0, m)
        def _main(i):
            for t in range(NSLOT):
                step(w0 + NSLOT * i + t, slots[t], False, False)

        step(nwin - 2, (nwin - 2) % NSLOT, False, True)
        step(nwin - 1, (nwin - 1) % NSLOT, False, True)
        for b in range(NSLOT):
            wait_scat(b)

        plsc.subcore_barrier()
        pltpu.sync_copy(acc.at[pl.ds(row0, npw)],
                        out_hbm.at[c].at[pl.ds(row0, npw)])

    return k(new_edge, receivers, zeros)


def _node_body(*refs):
    nf = refs[0]
    parts = refs[1:1 + NCHUNK]
    (w1a, w1b, b1, w2, b2, w3, b3, gam, bet, out) = refs[1 + NCHUNK:]
    nfb = nf[...]
    ssum = parts[0][0] + parts[0][1]
    for pr in parts[1:]:
        ssum = ssum + pr[0] + pr[1]
    h = (jnp.dot(nfb, w1a[...], preferred_element_type=jnp.float32)
         + jnp.dot(ssum, w1b[...], preferred_element_type=jnp.float32)
         + b1[...])
    h = jnp.maximum(h, 0.0)
    h = jnp.dot(h, w2[...], preferred_element_type=jnp.float32) + b2[...]
    h = jnp.maximum(h, 0.0)
    h = jnp.dot(h, w3[...], preferred_element_type=jnp.float32) + b3[...]
    mu = jnp.mean(h, axis=-1, keepdims=True)
    hc = h - mu
    var = jnp.mean(hc * hc, axis=-1, keepdims=True)
    ln = gam[...] * hc / jnp.sqrt(var + 1e-5) + bet[...]
    out[...] = ln + nfb


def _node_mlp(nf, parts, w1a, w1b, b1, w2, b2, w3, b3, gam, bet):
    wspec = pl.BlockSpec((D, D), lambda i: (0, 0))
    bspec = pl.BlockSpec((1, D), lambda i: (0, 0))
    rspec = pl.BlockSpec((NBLK, D), lambda i: (i, 0))
    pspec = pl.BlockSpec((NC, NBLK, D), lambda i: (0, i, 0))
    return pl.pallas_call(
        _node_body,
        grid=(N // NBLK,),
        in_specs=[rspec] + [pspec] * NCHUNK + [wspec, wspec, bspec, wspec,
                                               bspec, wspec, bspec, bspec,
                                               bspec],
        out_specs=pl.BlockSpec((NBLK, D), lambda i: (i, 0)),
        out_shape=jax.ShapeDtypeStruct((N, D), jnp.float32),
    )(nf, *parts, w1a, w1b, b1, w2, b2, w3, b3, gam, bet)


def kernel(node_features, edge_features, senders, receivers,
           eW1, eb1, eW2, eb2, eW3, eb3, eg, ebt,
           nW1, nb1, nW2, nb2, nW3, nb3, ng, nbt):
    nf = node_features.reshape(N, D)
    ef = edge_features.reshape(E, D)

    p, q = _compute_pq(nf, eW1[:D], eW1[D:2 * D], eb1.reshape(1, D))
    zeros = jnp.zeros((N, D), jnp.float32)

    w1c = eW1[2 * D:]
    eb2r = eb2.reshape(1, D)
    eb3r = eb3.reshape(1, D)
    egr = eg.reshape(1, D)
    ebtr = ebt.reshape(1, D)

    neo = None
    parts = []
    for chunk in range(NCHUNK):
        g = _gather_add(p, q, senders, receivers, chunk)
        neo, ne = _edge_mlp(g, ef, w1c, eW2, eb2r, eW3, eb3r, egr, ebtr,
                            chunk, neo)
        parts.append(_scatter_add(ne, receivers, zeros, chunk))

    nn = _node_mlp(nf, parts, nW1[:D], nW1[D:2 * D], nb1.reshape(1, D),
                   nW2, nb2.reshape(1, D), nW3, nb3.reshape(1, D),
                   ng.reshape(1, D), nbt.reshape(1, D))
    return nn.reshape(1, N, D), neo.reshape(1, E, D)
